# baseline (speedup 1.0000x reference)
import jax
import jax.numpy as jnp
from jax.experimental import pallas as pl

D_MODEL = 1024
D_MEM = 128
H = 8
DMT = H * D_MEM
TOKENS_PER_BLOCK = 2048

_DN_T = (((1,), (1,)), ((), ()))


def _sdtm_block(x_ref, we_ref, wd_ref, gw_ref, gb_ref, mc_ref, o_ref):
    x = x_ref[...]
    zp = jax.lax.dot_general(x, we_ref[...], _DN_T)
    z = 0.5 * zp * (1.0 + jax.lax.erf(zp * 0.7071067811865476))
    alpha = jax.nn.sigmoid(jax.lax.dot_general(x, gw_ref[...], _DN_T) + gb_ref[...])
    parts = []
    for h in range(H):
        z_h = z[:, h * D_MEM:(h + 1) * D_MEM]
        sd = jnp.dot(z_h, mc_ref[h])
        parts.append(sd[:, :D_MEM] + alpha[:, h:h + 1] * sd[:, D_MEM:])
    r = jnp.concatenate(parts, axis=1)
    o_ref[...] = jax.lax.dot_general(r, wd_ref[...], _DN_T)


def kernel(x, W_enc, W_dec, gate_W, gate_b, inject_scale, M_fast, M_slow):
    B, S, _ = x.shape
    N = B * S
    xf = x.reshape(N, D_MODEL)
    scale = jax.nn.sigmoid(inject_scale)
    gb = gate_b.reshape(1, H)
    mc = scale * jnp.concatenate([M_slow, M_fast - M_slow], axis=2)
    T = TOKENS_PER_BLOCK
    out = pl.pallas_call(
        _sdtm_block,
        grid=(N // T,),
        in_specs=[
            pl.BlockSpec((T, D_MODEL), lambda i: (i, 0)),
            pl.BlockSpec((DMT, D_MODEL), lambda i: (0, 0)),
            pl.BlockSpec((D_MODEL, DMT), lambda i: (0, 0)),
            pl.BlockSpec((H, D_MODEL), lambda i: (0, 0)),
            pl.BlockSpec((1, H), lambda i: (0, 0)),
            pl.BlockSpec((H, D_MEM, 2 * D_MEM), lambda i: (0, 0, 0)),
        ],
        out_specs=pl.BlockSpec((T, D_MODEL), lambda i: (i, 0)),
        out_shape=jax.ShapeDtypeStruct((N, D_MODEL), jnp.float32),
    )(xf, W_enc, W_dec, gate_W, gb, mc)
    return out.reshape(B, S, D_MODEL)


# speedup vs baseline: 1.0151x; 1.0151x over previous
import jax
import jax.numpy as jnp
from jax.experimental import pallas as pl

D_MODEL = 1024
D_MEM = 128
H = 8
DMT = H * D_MEM
TOKENS_PER_BLOCK = 1024

_DN_T = (((1,), (1,)), ((), ()))


def _sdtm_block(x_ref, we_ref, wd_ref, gw_ref, gb_ref, mc_ref, o_ref):
    x = x_ref[...]
    zp = jax.lax.dot_general(x, we_ref[...], _DN_T)
    z = 0.5 * zp * (1.0 + jax.lax.erf(zp * 0.7071067811865476))
    alpha = jax.nn.sigmoid(jax.lax.dot_general(x, gw_ref[...], _DN_T) + gb_ref[...])
    parts = []
    for h in range(H):
        z_h = z[:, h * D_MEM:(h + 1) * D_MEM]
        sd = jnp.dot(z_h, mc_ref[h])
        parts.append(sd[:, :D_MEM] + alpha[:, h:h + 1] * sd[:, D_MEM:])
    r = jnp.concatenate(parts, axis=1)
    o_ref[...] = jax.lax.dot_general(r, wd_ref[...], _DN_T)


def kernel(x, W_enc, W_dec, gate_W, gate_b, inject_scale, M_fast, M_slow):
    B, S, _ = x.shape
    N = B * S
    xf = x.reshape(N, D_MODEL)
    scale = jax.nn.sigmoid(inject_scale)
    gb = gate_b.reshape(1, H)
    mc = scale * jnp.concatenate([M_slow, M_fast - M_slow], axis=2)
    T = TOKENS_PER_BLOCK
    out = pl.pallas_call(
        _sdtm_block,
        grid=(N // T,),
        in_specs=[
            pl.BlockSpec((T, D_MODEL), lambda i: (i, 0)),
            pl.BlockSpec((DMT, D_MODEL), lambda i: (0, 0)),
            pl.BlockSpec((D_MODEL, DMT), lambda i: (0, 0)),
            pl.BlockSpec((H, D_MODEL), lambda i: (0, 0)),
            pl.BlockSpec((1, H), lambda i: (0, 0)),
            pl.BlockSpec((H, D_MEM, 2 * D_MEM), lambda i: (0, 0, 0)),
        ],
        out_specs=pl.BlockSpec((T, D_MODEL), lambda i: (i, 0)),
        out_shape=jax.ShapeDtypeStruct((N, D_MODEL), jnp.float32),
    )(xf, W_enc, W_dec, gate_W, gb, mc)
    return out.reshape(B, S, D_MODEL)
